# baseline (speedup 1.0000x reference)
import functools

import jax
import jax.numpy as jnp
from jax import lax
from jax.experimental import pallas as pl
from jax.experimental.pallas import tpu as pltpu
from jax.experimental.pallas import tpu_sc as plsc

_NC = 2
_NS = 16
_NW = _NC * _NS

_NBUF = 2
_LP = 64

_mesh = plsc.VectorSubcoreMesh(core_axis_name="c", subcore_axis_name="s")
_params = pltpu.CompilerParams(
    use_tc_tiling_on_sc=False, needs_layout_passes=False)


def _pipeline(L, RPW, idx_list, table, out, half, bufs, gsem, wsem):
    D = table.shape[-1]
    wid = lax.axis_index("s") * _NC + lax.axis_index("c")
    base = wid * RPW
    gathers = [None] * _NBUF
    writes = [None] * _NBUF

    def start(t):
        b = t % _NBUF
        gathers[b] = pltpu.async_copy(
            table.at[idx_list.at[pl.ds(t * RPW, RPW)]], bufs.at[b],
            gsem.at[b])

    def retire(t):
        b = t % _NBUF
        gathers[b].wait()
        writes[b] = pltpu.async_copy(
            bufs.at[b], out.at[t, pl.ds(base, RPW), pl.ds(half * D, D)],
            wsem.at[b])

    for t in range(L):
        b = t % _NBUF
        if t >= _NBUF:
            writes[b].wait()
        start(t)
        if t >= 1:
            retire(t - 1)
    retire(L - 1)
    for b in range(_NBUF):
        writes[b].wait()


@functools.lru_cache(maxsize=None)
def _build_mat(N, L, D):
    RPW = N // _NW
    FPW = RPW * L

    @functools.partial(
        pl.kernel, mesh=_mesh, compiler_params=_params, out_type=(),
        scratch_types=[
            pltpu.VMEM((RPW,), jnp.int32),
            pltpu.VMEM((FPW,), jnp.int32),
            pltpu.VMEM((_NBUF, RPW, D), jnp.float32),
            pltpu.SemaphoreType.DMA((_NBUF,)),
            pltpu.SemaphoreType.DMA((_NBUF,)),
        ],
    )
    def mat_kernel(mat_hbm, idx_hbm, out_ref,
                   idx_v, mat_idx, bufs, gsem, wsem):
        wid = lax.axis_index("s") * _NC + lax.axis_index("c")
        base = wid * RPW
        pltpu.sync_copy(idx_hbm.at[pl.ds(base, RPW)], idx_v)

        def build(k, _):
            q = k * 16 + lax.iota(jnp.int32, 16)
            t = q >> 7
            j = q & (RPW - 1)
            srow = plsc.load_gather(idx_v, [j])
            mat_idx[pl.ds(k * 16, 16)] = srow * L + t
            return 0
        lax.fori_loop(0, FPW // 16, build, 0)
        _pipeline(L, RPW, mat_idx, mat_hbm, out_ref, 0, bufs, gsem, wsem)

    return mat_kernel


@functools.lru_cache(maxsize=None)
def _build_emb(N, L, D):
    RPW = N // _NW
    FPW = RPW * L

    @functools.partial(
        pl.kernel, mesh=_mesh, compiler_params=_params, out_type=(),
        scratch_types=[
            pltpu.VMEM((RPW,), jnp.int32),
            pltpu.VMEM((RPW, _LP), jnp.int32),
            pltpu.VMEM((FPW,), jnp.int32),
            pltpu.VMEM((_NBUF, RPW, D), jnp.float32),
            pltpu.SemaphoreType.DMA,
            pltpu.SemaphoreType.DMA((_NBUF,)),
            pltpu.SemaphoreType.DMA((_NBUF,)),
        ],
    )
    def emb_kernel(ct_hbm, emb_hbm, idx_hbm, out_ref,
                   idx_v, ct_v, ct_idx, bufs, sem0, gsem, wsem):
        wid = lax.axis_index("s") * _NC + lax.axis_index("c")
        base = wid * RPW
        pltpu.sync_copy(idx_hbm.at[pl.ds(base, RPW)], idx_v)
        pltpu.async_copy(ct_hbm.at[idx_v], ct_v, sem0).wait()

        def build(k, _):
            q = k * 16 + lax.iota(jnp.int32, 16)
            t = q >> 7
            j = q & (RPW - 1)
            ct_idx[pl.ds(k * 16, 16)] = plsc.load_gather(ct_v, [j, t])
            return 0
        lax.fori_loop(0, FPW // 16, build, 0)
        _pipeline(L, RPW, ct_idx, emb_hbm, out_ref, 1, bufs, gsem, wsem)

    return emb_kernel


def kernel(matrix, length, core_terms, emb):
    G, B, L, D = matrix.shape
    N = G * B
    V = emb.shape[0]

    length_flat = length.reshape(-1)
    idx_sort = jnp.argsort(-length_flat).astype(jnp.int32)
    idx_unsort = jnp.argsort(idx_sort).astype(jnp.int32)
    length_sorted = jnp.take(length_flat, idx_sort)

    mat_flat = lax.optimization_barrier(
        matrix.astype(jnp.float32).reshape(-1))
    emb_flat = lax.optimization_barrier(emb.astype(jnp.float32).reshape(-1))
    mat = mat_flat.reshape(N * L, D)
    emb2 = emb_flat.reshape(V, D)
    ct = jnp.pad(core_terms.reshape(N, L), ((0, 0), (0, _LP - L)))

    x_ref = jax.new_ref(jnp.zeros((L, N, 2 * D), jnp.float32))
    _build_mat(N, L, D)(mat, idx_sort, x_ref)
    _build_emb(N, L, D)(ct, emb2, idx_sort, x_ref)
    x = jnp.transpose(x_ref[...], (1, 0, 2))
    return x, length_sorted, idx_unsort


# speedup vs baseline: 1.0283x; 1.0283x over previous
import functools

import jax
import jax.numpy as jnp
from jax import lax
from jax.experimental import pallas as pl
from jax.experimental.pallas import tpu as pltpu
from jax.experimental.pallas import tpu_sc as plsc

_NC = 2
_NS = 16
_NW = _NC * _NS

_NBUF = 2
_LP = 64

_mesh = plsc.VectorSubcoreMesh(core_axis_name="c", subcore_axis_name="s")
_params = pltpu.CompilerParams(
    use_tc_tiling_on_sc=False, needs_layout_passes=False)


def _pipeline(L, RPW, idx_list, table, out, half, bufs, gsem, wsem):
    D = table.shape[-1]
    wid = lax.axis_index("s") * _NC + lax.axis_index("c")
    base = wid * RPW
    gathers = [None] * _NBUF
    writes = [None] * _NBUF

    def start(t):
        b = t % _NBUF
        gathers[b] = pltpu.async_copy(
            table.at[idx_list.at[pl.ds(t * RPW, RPW)]], bufs.at[b],
            gsem.at[b])

    def retire(t):
        b = t % _NBUF
        gathers[b].wait()
        writes[b] = pltpu.async_copy(
            bufs.at[b], out.at[t, pl.ds(base, RPW), pl.ds(half * D, D)],
            wsem.at[b])

    for t in range(L):
        b = t % _NBUF
        if t >= _NBUF:
            writes[b].wait()
        start(t)
        if t >= 1:
            retire(t - 1)
    retire(L - 1)
    for b in range(_NBUF):
        writes[b].wait()


@functools.lru_cache(maxsize=None)
def _build_mat(N, L, D):
    RPW = N // _NW
    FPW = RPW * L

    @functools.partial(
        pl.kernel, mesh=_mesh, compiler_params=_params,
        out_type=jax.ShapeDtypeStruct((L, N, 2 * D), jnp.float32),
        scratch_types=[
            pltpu.VMEM((RPW,), jnp.int32),
            pltpu.VMEM((FPW,), jnp.int32),
            pltpu.VMEM((_NBUF, RPW, D), jnp.float32),
            pltpu.SemaphoreType.DMA((_NBUF,)),
            pltpu.SemaphoreType.DMA((_NBUF,)),
        ],
    )
    def mat_kernel(mat_hbm, idx_hbm, out_ref,
                   idx_v, mat_idx, bufs, gsem, wsem):
        wid = lax.axis_index("s") * _NC + lax.axis_index("c")
        base = wid * RPW
        pltpu.sync_copy(idx_hbm.at[pl.ds(base, RPW)], idx_v)

        def build(k, _):
            q = k * 16 + lax.iota(jnp.int32, 16)
            t = q >> 7
            j = q & (RPW - 1)
            srow = plsc.load_gather(idx_v, [j])
            mat_idx[pl.ds(k * 16, 16)] = srow * L + t
            return 0
        lax.fori_loop(0, FPW // 16, build, 0)
        _pipeline(L, RPW, mat_idx, mat_hbm, out_ref, 0, bufs, gsem, wsem)

    return mat_kernel


@functools.lru_cache(maxsize=None)
def _build_emb(N, L, D):
    RPW = N // _NW
    FPW = RPW * L

    @functools.partial(
        pl.kernel, mesh=_mesh, compiler_params=_params, out_type=(),
        scratch_types=[
            pltpu.VMEM((RPW,), jnp.int32),
            pltpu.VMEM((RPW, _LP), jnp.int32),
            pltpu.VMEM((FPW,), jnp.int32),
            pltpu.VMEM((_NBUF, RPW, D), jnp.float32),
            pltpu.SemaphoreType.DMA,
            pltpu.SemaphoreType.DMA((_NBUF,)),
            pltpu.SemaphoreType.DMA((_NBUF,)),
        ],
    )
    def emb_kernel(ct_hbm, emb_hbm, idx_hbm, out_ref,
                   idx_v, ct_v, ct_idx, bufs, sem0, gsem, wsem):
        wid = lax.axis_index("s") * _NC + lax.axis_index("c")
        base = wid * RPW
        pltpu.sync_copy(idx_hbm.at[pl.ds(base, RPW)], idx_v)
        pltpu.async_copy(ct_hbm.at[idx_v], ct_v, sem0).wait()

        def build(k, _):
            q = k * 16 + lax.iota(jnp.int32, 16)
            t = q >> 7
            j = q & (RPW - 1)
            ct_idx[pl.ds(k * 16, 16)] = plsc.load_gather(ct_v, [j, t])
            return 0
        lax.fori_loop(0, FPW // 16, build, 0)
        _pipeline(L, RPW, ct_idx, emb_hbm, out_ref, 1, bufs, gsem, wsem)

    return emb_kernel


def kernel(matrix, length, core_terms, emb):
    G, B, L, D = matrix.shape
    N = G * B
    V = emb.shape[0]

    length_flat = length.reshape(-1)
    idx_sort = jnp.argsort(-length_flat).astype(jnp.int32)
    idx_unsort = jnp.argsort(idx_sort).astype(jnp.int32)
    length_sorted = jnp.take(length_flat, idx_sort)

    mat_flat = lax.optimization_barrier(
        matrix.astype(jnp.float32).reshape(-1))
    emb_flat = lax.optimization_barrier(emb.astype(jnp.float32).reshape(-1))
    mat = mat_flat.reshape(N * L, D)
    emb2 = emb_flat.reshape(V, D)
    ct = jnp.pad(core_terms.reshape(N, L), ((0, 0), (0, _LP - L)))

    x_mat = _build_mat(N, L, D)(mat, idx_sort)
    x_ref = jax.new_ref(x_mat)
    _build_emb(N, L, D)(ct, emb2, idx_sort, x_ref)
    x = jnp.transpose(x_ref[...], (1, 0, 2))
    return x, length_sorted, idx_unsort


# speedup vs baseline: 1.0404x; 1.0118x over previous
import functools

import jax
import jax.numpy as jnp
from jax import lax
from jax.experimental import pallas as pl
from jax.experimental.pallas import tpu as pltpu
from jax.experimental.pallas import tpu_sc as plsc

_NC = 2
_NS = 16
_NW = _NC * _NS

_NBUF = 2
_LP = 64

_mesh = plsc.VectorSubcoreMesh(core_axis_name="c", subcore_axis_name="s")
_params = pltpu.CompilerParams(
    use_tc_tiling_on_sc=False, needs_layout_passes=False)


def _pipeline(L, RPW, idx_list, table, out, half, bufs, gsem, wsem):
    D = table.shape[-1]
    wid = lax.axis_index("s") * _NC + lax.axis_index("c")
    base = wid * RPW
    gathers = [None] * _NBUF
    writes = [None] * _NBUF

    def start(t):
        b = t % _NBUF
        gathers[b] = pltpu.async_copy(
            table.at[idx_list.at[pl.ds(t * RPW, RPW)]], bufs.at[b],
            gsem.at[b])

    def retire(t):
        b = t % _NBUF
        gathers[b].wait()
        writes[b] = pltpu.async_copy(
            bufs.at[b], out.at[t, pl.ds(base, RPW), pl.ds(half * D, D)],
            wsem.at[b])

    for t in range(L):
        b = t % _NBUF
        if t >= _NBUF:
            writes[b].wait()
        start(t)
        if t >= 1:
            retire(t - 1)
    retire(L - 1)
    for b in range(_NBUF):
        writes[b].wait()


@functools.lru_cache(maxsize=None)
def _build_mat(N, L, D):
    RPW = N // _NW
    FPW = RPW * L

    @functools.partial(
        pl.kernel, mesh=_mesh, compiler_params=_params,
        out_type=(
            jax.ShapeDtypeStruct((L, N, 2 * D), jnp.float32),
            jax.ShapeDtypeStruct((N * L,), jnp.int32),
            jax.ShapeDtypeStruct((N,), jnp.int32),
        ),
        scratch_types=[
            pltpu.VMEM((RPW,), jnp.int32),
            pltpu.VMEM((RPW, _LP), jnp.int32),
            pltpu.VMEM((N,), jnp.int32),
            pltpu.VMEM((RPW,), jnp.int32),
            pltpu.VMEM((FPW,), jnp.int32),
            pltpu.VMEM((FPW,), jnp.int32),
            pltpu.VMEM((_NBUF, RPW, D), jnp.float32),
            pltpu.SemaphoreType.DMA,
            pltpu.SemaphoreType.DMA((_NBUF,)),
            pltpu.SemaphoreType.DMA((_NBUF,)),
        ],
    )
    def mat_kernel(mat_hbm, ct_hbm, len_hbm, idx_hbm,
                   out_ref, ctidx_hbm, lsort_hbm,
                   idx_v, ct_v, len_all, lsort_v, mat_idx, ct_idx,
                   bufs, sem0, gsem, wsem):
        wid = lax.axis_index("s") * _NC + lax.axis_index("c")
        base = wid * RPW
        pltpu.sync_copy(idx_hbm.at[pl.ds(base, RPW)], idx_v)
        pltpu.async_copy(ct_hbm.at[idx_v], ct_v, sem0).wait()

        pltpu.sync_copy(len_hbm, len_all)
        for j in range(RPW // 16):
            idx16 = idx_v[pl.ds(j * 16, 16)]
            lsort_v[pl.ds(j * 16, 16)] = plsc.load_gather(len_all, [idx16])
        pltpu.sync_copy(lsort_v, lsort_hbm.at[pl.ds(base, RPW)])

        def build(k, _):
            q = k * 16 + lax.iota(jnp.int32, 16)
            t = q >> 7
            j = q & (RPW - 1)
            srow = plsc.load_gather(idx_v, [j])
            mat_idx[pl.ds(k * 16, 16)] = srow * L + t
            ct_idx[pl.ds(k * 16, 16)] = plsc.load_gather(ct_v, [j, t])
            return 0
        lax.fori_loop(0, FPW // 16, build, 0)
        pltpu.sync_copy(ct_idx, ctidx_hbm.at[pl.ds(wid * FPW, FPW)])

        _pipeline(L, RPW, mat_idx, mat_hbm, out_ref, 0, bufs, gsem, wsem)

    return mat_kernel


@functools.lru_cache(maxsize=None)
def _build_emb(N, L, D):
    RPW = N // _NW
    FPW = RPW * L

    @functools.partial(
        pl.kernel, mesh=_mesh, compiler_params=_params, out_type=(),
        scratch_types=[
            pltpu.VMEM((FPW,), jnp.int32),
            pltpu.VMEM((_NBUF, RPW, D), jnp.float32),
            pltpu.SemaphoreType.DMA((_NBUF,)),
            pltpu.SemaphoreType.DMA((_NBUF,)),
        ],
    )
    def emb_kernel(ctidx_hbm, emb_hbm, out_ref,
                   ct_idx, bufs, gsem, wsem):
        wid = lax.axis_index("s") * _NC + lax.axis_index("c")
        pltpu.sync_copy(ctidx_hbm.at[pl.ds(wid * FPW, FPW)], ct_idx)
        _pipeline(L, RPW, ct_idx, emb_hbm, out_ref, 1, bufs, gsem, wsem)

    return emb_kernel


def kernel(matrix, length, core_terms, emb):
    G, B, L, D = matrix.shape
    N = G * B
    V = emb.shape[0]

    length_flat = length.reshape(-1)
    idx_sort = jnp.argsort(-length_flat).astype(jnp.int32)
    idx_unsort = jnp.argsort(idx_sort).astype(jnp.int32)

    mat_flat = lax.optimization_barrier(
        matrix.astype(jnp.float32).reshape(-1))
    emb_flat = lax.optimization_barrier(emb.astype(jnp.float32).reshape(-1))
    mat = mat_flat.reshape(N * L, D)
    emb2 = emb_flat.reshape(V, D)
    ct = jnp.pad(core_terms.reshape(N, L), ((0, 0), (0, _LP - L)))

    x_mat, ct_idx_sorted, length_sorted = _build_mat(N, L, D)(
        mat, ct, length_flat, idx_sort)
    x_ref = jax.new_ref(x_mat)
    _build_emb(N, L, D)(ct_idx_sorted, emb2, x_ref)
    x = jnp.transpose(x_ref[...], (1, 0, 2))
    return x, length_sorted, idx_unsort
